# baseline (speedup 1.0000x reference)
import functools

import jax
import jax.numpy as jnp
from jax import lax
from jax.experimental import pallas as pl
from jax.experimental.pallas import tpu as pltpu
from jax.experimental.pallas import tpu_sc as plsc

DIM = 128
BATCH = 4096
HIST = 50
NCORES = 2
NSUB = 16
NW = NCORES * NSUB
BPW = BATCH // NW
NACC = 7
LANES = 16

_SELU_ALPHA = 1.6732632423543772
_SELU_SCALE = 1.0507009873554805


def _sc_pool(idx_t, table):
    mesh = plsc.VectorSubcoreMesh(
        core_axis_name="c", subcore_axis_name="s",
        num_cores=NCORES, num_subcores=NSUB,
    )

    @functools.partial(
        pl.kernel,
        out_type=jax.ShapeDtypeStruct((BATCH, DIM), jnp.float32),
        mesh=mesh,
        scratch_types=[
            pltpu.VMEM((HIST * BPW,), jnp.int32),
        ]
        + [pltpu.VMEM((BPW, DIM), jnp.float32)] * NACC
        + [pltpu.SemaphoreType.DMA] * NACC,
    )
    def pool(idx_hbm, table_hbm, out_hbm, idx_v, *ring):
        bufs = ring[:NACC]
        sems = ring[NACC:]
        wid = lax.axis_index("c") * NSUB + lax.axis_index("s")
        base = wid * BPW
        pltpu.sync_copy(idx_hbm.at[pl.ds(wid * (HIST * BPW), HIST * BPW)], idx_v)

        pending = [None] * NACC
        for h in range(HIST):
            b = h % NACC
            if pending[b] is not None:
                pending[b].wait()
            pending[b] = pltpu.async_copy(
                table_hbm.at[idx_v.at[pl.ds(h * BPW, BPW)]],
                bufs[b], sems[b], add=(h >= NACC),
            )
        for b in range(NACC):
            pending[b].wait()

        def reduce_row(r, carry):
            for d in range(DIM // LANES):
                sl = pl.ds(d * LANES, LANES)
                acc = bufs[0][r, sl]
                for b in range(1, NACC):
                    acc = acc + bufs[b][r, sl]
                bufs[0][r, sl] = acc
            return carry

        lax.fori_loop(0, BPW, reduce_row, 0, unroll=2)
        pltpu.sync_copy(bufs[0], out_hbm.at[pl.ds(base, BPW)])

    return pool(idx_t, table)


def _selu_linear(x, wT, b2):
    def body(x_ref, w_ref, b_ref, o_ref):
        v = x_ref[...]
        v = _SELU_SCALE * jnp.where(v > 0, v, _SELU_ALPHA * (jnp.exp(v) - 1.0))
        o_ref[...] = (
            jnp.dot(v, w_ref[...], preferred_element_type=jnp.float32) + b_ref[...]
        )

    blk = 512
    return pl.pallas_call(
        body,
        out_shape=jax.ShapeDtypeStruct((BATCH, DIM), jnp.float32),
        grid=(BATCH // blk,),
        in_specs=[
            pl.BlockSpec((blk, DIM), lambda i: (i, 0)),
            pl.BlockSpec((DIM, DIM), lambda i: (0, 0)),
            pl.BlockSpec((1, DIM), lambda i: (0, 0)),
        ],
        out_specs=pl.BlockSpec((blk, DIM), lambda i: (i, 0)),
    )(x, wT, b2)


def kernel(input_text, table, W, b):
    idx = input_text.reshape(BATCH, HIST).astype(jnp.int32)
    idx_t = idx.reshape(NW, BPW, HIST).transpose(0, 2, 1).reshape(-1)
    pooled = _sc_pool(idx_t, table)
    return _selu_linear(pooled, W.T, b.reshape(1, DIM))


# speedup vs baseline: 1.0586x; 1.0586x over previous
import functools

import jax
import jax.numpy as jnp
from jax import lax
from jax.experimental import pallas as pl
from jax.experimental.pallas import tpu as pltpu
from jax.experimental.pallas import tpu_sc as plsc

DIM = 128
BATCH = 4096
HIST = 50
HPAD = 56
NCORES = 2
NSUB = 16
NW = NCORES * NSUB
BPW = BATCH // NW
NBUF = 8
LANES = 16

_SELU_ALPHA = 1.6732632423543772
_SELU_SCALE = 1.0507009873554805


def _sc_pool(idx_flat, table):
    mesh = plsc.VectorSubcoreMesh(
        core_axis_name="c", subcore_axis_name="s",
        num_cores=NCORES, num_subcores=NSUB,
    )

    GROUPS = BPW // 4
    RING = NBUF // 2
    SPLIT = 96
    SEGS = [
        [(0, 0, HIST)],
        [(0, HIST, SPLIT - HIST), (1, 0, 2 * HIST - SPLIT)],
        [(1, 2 * HIST - SPLIT, HIST)],
        [(1, 3 * HIST - SPLIT, HIST)],
    ]

    @functools.partial(
        pl.kernel,
        out_type=jax.ShapeDtypeStruct((BATCH, DIM), jnp.float32),
        mesh=mesh,
        scratch_types=[
            pltpu.VMEM((BPW * HIST,), jnp.int32),
            pltpu.VMEM((BPW, DIM), jnp.float32),
        ]
        + [pltpu.VMEM((SPLIT, DIM), jnp.float32),
           pltpu.VMEM((200 - SPLIT, DIM), jnp.float32)] * RING
        + [pltpu.SemaphoreType.DMA] * NBUF,
    )
    def pool(idx_hbm, table_hbm, out_hbm, idx_v, outbuf, *ring):
        bufs = ring[:NBUF]
        sems = ring[NBUF:]
        wid = lax.axis_index("c") * NSUB + lax.axis_index("s")
        base = wid * BPW
        pltpu.sync_copy(idx_hbm.at[pl.ds(wid * (BPW * HIST), BPW * HIST)], idx_v)

        def dma(g, k, half):
            off = 200 * g + (SPLIT if half else 0)
            ln = (200 - SPLIT) if half else SPLIT
            return pltpu.make_async_copy(
                table_hbm.at[idx_v.at[pl.ds(off, ln)]],
                bufs[2 * k + half], sems[2 * k + half],
            )

        def reduce_group(g, k):
            for r in range(4):
                segs = [(bufs[2 * k + half], st, ln) for half, st, ln in SEGS[r]]
                buf0, st0, _ = segs[0]
                accs = tuple(buf0[st0, pl.ds(d * LANES, LANES)] for d in range(8))
                for buf, st, ln in segs:
                    lo = st + 1 if buf is buf0 and st == st0 else st

                    def inner(h, a8, _buf=buf):
                        return tuple(
                            a + _buf[h, pl.ds(d * LANES, LANES)]
                            for d, a in enumerate(a8)
                        )

                    accs = lax.fori_loop(lo, st + ln, inner, accs, unroll=7)
                for d in range(8):
                    outbuf[4 * g + r, pl.ds(d * LANES, LANES)] = accs[d]

        for k in range(RING):
            dma(k, k, 0).start()
            dma(k, k, 1).start()

        def step(s, carry):
            g0 = s * RING
            for k in range(RING):
                g = g0 + k
                dma(g, k, 0).wait()
                dma(g, k, 1).wait()
                reduce_group(g, k)

                @pl.when(g + RING < GROUPS)
                def _():
                    dma(g + RING, k, 0).start()
                    dma(g + RING, k, 1).start()

            return carry

        lax.fori_loop(0, GROUPS // RING, step, 0)
        pltpu.sync_copy(outbuf, out_hbm.at[pl.ds(base, BPW)])

    return pool(idx_flat, table)


def _selu_linear(x, wT, b2):
    def body(x_ref, w_ref, b_ref, o_ref):
        v = x_ref[...]
        v = _SELU_SCALE * jnp.where(v > 0, v, _SELU_ALPHA * (jnp.exp(v) - 1.0))
        o_ref[...] = (
            jnp.dot(v, w_ref[...], preferred_element_type=jnp.float32) + b_ref[...]
        )

    return pl.pallas_call(
        body,
        out_shape=jax.ShapeDtypeStruct((BATCH, DIM), jnp.float32),
    )(x, wT, b2)


def kernel(input_text, table, W, b):
    idx = input_text.reshape(BATCH, HIST).astype(jnp.int32)
    pooled = _sc_pool(idx.reshape(-1), table)
    return _selu_linear(pooled, W.T, b.reshape(1, DIM))


# speedup vs baseline: 1.1951x; 1.1289x over previous
import functools

import jax
import jax.numpy as jnp
from jax import lax
from jax.experimental import pallas as pl
from jax.experimental.pallas import tpu as pltpu
from jax.experimental.pallas import tpu_sc as plsc

DIM = 128
BATCH = 4096
HIST = 50
HPAD = 56
NCORES = 2
NSUB = 16
NW = NCORES * NSUB
BPW = BATCH // NW
NBUF = 8
LANES = 16

_SELU_ALPHA = 1.6732632423543772
_SELU_SCALE = 1.0507009873554805


def _sc_pool(idx_flat, table):
    mesh = plsc.VectorSubcoreMesh(
        core_axis_name="c", subcore_axis_name="s",
        num_cores=NCORES, num_subcores=NSUB,
    )

    @functools.partial(
        pl.kernel,
        out_type=jax.ShapeDtypeStruct((BATCH, DIM), jnp.float32),
        mesh=mesh,
        scratch_types=[
            pltpu.VMEM((BPW * HPAD,), jnp.int32),
            pltpu.VMEM((BPW, DIM), jnp.float32),
        ]
        + [pltpu.VMEM((HIST, DIM), jnp.float32)] * NBUF
        + [pltpu.SemaphoreType.DMA] * NBUF,
    )
    def pool(idx_hbm, table_hbm, out_hbm, idx_v, outbuf, *ring):
        bufs = ring[:NBUF]
        sems = ring[NBUF:]
        wid = lax.axis_index("c") * NSUB + lax.axis_index("s")
        base = wid * BPW
        pltpu.sync_copy(idx_hbm.at[pl.ds(wid * (BPW * HPAD), BPW * HPAD)], idx_v)

        def dma(row, slot):
            return pltpu.make_async_copy(
                table_hbm.at[idx_v.at[pl.ds(row * HPAD, HIST)]],
                bufs[slot], sems[slot],
            )

        def reduce_row(row, slot):
            buf = bufs[slot]
            accs = tuple(buf[0, pl.ds(d * LANES, LANES)] for d in range(8))

            def inner(h, a8):
                return tuple(
                    a + buf[h, pl.ds(d * LANES, LANES)]
                    for d, a in enumerate(a8)
                )

            accs = lax.fori_loop(1, HIST, inner, accs, unroll=7)
            for d in range(8):
                outbuf[row, pl.ds(d * LANES, LANES)] = accs[d]

        for k in range(NBUF):
            dma(k, k).start()

        def step(s, carry):
            r0 = s * NBUF
            for k in range(NBUF):
                row = r0 + k
                dma(row, k).wait()
                reduce_row(row, k)

                @pl.when(row + NBUF < BPW)
                def _():
                    dma(row + NBUF, k).start()

            return carry

        lax.fori_loop(0, BPW // NBUF, step, 0)
        pltpu.sync_copy(outbuf, out_hbm.at[pl.ds(base, BPW)])

    return pool(idx_flat, table)


def _selu_linear(x, wT, b2):
    def body(x_ref, w_ref, b_ref, o_ref):
        v = x_ref[...]
        v = _SELU_SCALE * jnp.where(v > 0, v, _SELU_ALPHA * (jnp.exp(v) - 1.0))
        o_ref[...] = (
            jnp.dot(v, w_ref[...], preferred_element_type=jnp.float32) + b_ref[...]
        )

    return pl.pallas_call(
        body,
        out_shape=jax.ShapeDtypeStruct((BATCH, DIM), jnp.float32),
    )(x, wT, b2)


def kernel(input_text, table, W, b):
    idx = input_text.reshape(BATCH, HIST).astype(jnp.int32)
    idx = jnp.pad(idx, ((0, 0), (0, HPAD - HIST)))
    pooled = _sc_pool(idx.reshape(-1), table)
    return _selu_linear(pooled, W.T, b.reshape(1, DIM))
